# baseline (speedup 1.0000x reference)
import functools

import jax
import jax.numpy as jnp
from jax import lax
from jax.experimental import pallas as pl
from jax.experimental.pallas import tpu as pltpu
from jax.experimental.pallas import tpu_sc as plsc

VOCAB = 1000000
D = 32
OUT = 16
B = 16384
SEQ = 50
EPS = 1e-5 * SEQ * SEQ

NC = 2
NS = 16
NW = NC * NS
BPW = B // NW
CHUNK = 128
NCH = BPW // CHUNK


VCHUNK = 65536
TROWS = VCHUNK // 4
FGRID = (VOCAB + VCHUNK - 1) // VCHUNK
VPAD = FGRID * VCHUNK
VSH = VCHUNK.bit_length() - 1
TSH = TROWS.bit_length() - 1


def _fmt_body(tt_ref, o_ref):
    a = tt_ref[...]
    b = jnp.concatenate(
        [a[:, q * TROWS:(q + 1) * TROWS] for q in range(4)], axis=0)
    o_ref[...] = b.T


def _fmt(tableT):
    return pl.pallas_call(
        _fmt_body,
        grid=(FGRID,),
        in_specs=[pl.BlockSpec((D, VCHUNK), lambda i: (0, i))],
        out_specs=pl.BlockSpec((TROWS, 128), lambda i: (i, 0)),
        out_shape=jax.ShapeDtypeStruct((VPAD // 4, 128), jnp.float32),
        compiler_params=pltpu.CompilerParams(
            vmem_limit_bytes=100 * 1024 * 1024),
    )(tableT)


def _sc_pool_body(table_hbm, xp_hbm, out_hbm, idx_v, acc_v, isem, sem):
    wid = lax.axis_index("s") * NC + lax.axis_index("c")

    idx_cp = pltpu.async_copy(xp_hbm.at[:, pl.ds(wid * BPW, BPW)], idx_v,
                              isem)

    def _zero_row(i, _):
        acc_v[i, pl.ds(0, 16)] = jnp.zeros((16,), jnp.float32)
        acc_v[i, pl.ds(16, 16)] = jnp.zeros((16,), jnp.float32)
        return 0
    lax.fori_loop(0, BPW, _zero_row, 0)
    idx_cp.wait()

    for c in range(NCH):
        def _fire(j, _, c=c):
            pltpu.async_copy(
                table_hbm.at[idx_v.at[j, pl.ds(c * CHUNK, CHUNK)]],
                acc_v.at[pl.ds(c * CHUNK, CHUNK)], sem, add=True)
            return 0
        lax.fori_loop(0, SEQ, _fire, 0)

    def _drain(t, _):
        pltpu.make_async_copy(table_hbm.at[idx_v.at[0, pl.ds(0, CHUNK)]],
                              acc_v.at[pl.ds(0, CHUNK)], sem).wait()
        return 0
    lax.fori_loop(0, SEQ * NCH, _drain, 0)

    pltpu.sync_copy(acc_v, out_hbm.at[pl.ds(wid * BPW, BPW)])


def _sc_pool(t4, x):
    mesh = plsc.VectorSubcoreMesh(core_axis_name="c", subcore_axis_name="s")
    return pl.kernel(
        _sc_pool_body,
        out_type=jax.ShapeDtypeStruct((B, D), jnp.float32),
        mesh=mesh,
        scratch_types=[
            pltpu.VMEM((SEQ, BPW), jnp.int32),
            pltpu.VMEM((BPW, D), jnp.float32),
            pltpu.SemaphoreType.DMA,
            pltpu.SemaphoreType.DMA,
        ],
        compiler_params=pltpu.CompilerParams(use_tc_tiling_on_sc=False,
                                             needs_layout_passes=False),
    )(t4, x)


def _head_body(s_ref, g_ref, be_ref, wt_ref, b_ref, o_ref):
    s = s_ref[...]
    mu = jnp.mean(s, axis=1, keepdims=True)
    var = jnp.mean((s - mu) ** 2, axis=1, keepdims=True)
    h = (s - mu) * lax.rsqrt(var + EPS) * g_ref[...] + be_ref[...]
    h = jnp.maximum(h, 0.0)
    o = jnp.dot(h, wt_ref[...],
                preferred_element_type=jnp.float32) + b_ref[...]
    o_ref[...] = o.T


def _head(sums, gamma, beta, Wt, bias):
    blk = 4096
    return pl.pallas_call(
        _head_body,
        grid=(B // blk,),
        in_specs=[
            pl.BlockSpec((blk, D), lambda i: (i, 0)),
            pl.BlockSpec((1, D), lambda i: (0, 0)),
            pl.BlockSpec((1, D), lambda i: (0, 0)),
            pl.BlockSpec((D, OUT), lambda i: (0, 0)),
            pl.BlockSpec((1, OUT), lambda i: (0, 0)),
        ],
        out_specs=pl.BlockSpec((OUT, blk), lambda i: (0, i)),
        out_shape=jax.ShapeDtypeStruct((OUT, B), jnp.float32),
    )(sums, gamma, beta, Wt, bias)


def kernel(x, table, ln_gamma, ln_beta, W, b):
    t4 = _fmt(table.T)
    xt = x.T
    xp = (((xt >> VSH) << VSH) + ((xt & (TROWS - 1)) << 2)
          + ((xt & (VCHUNK - 1)) >> TSH))
    sums = _sc_pool(t4.reshape(VPAD, D), xp)
    return _head(sums, ln_gamma.reshape(1, D), ln_beta.reshape(1, D),
                 W.T, b.reshape(1, OUT)).T


# speedup vs baseline: 1.0003x; 1.0003x over previous
import jax
import jax.numpy as jnp
from jax import lax
from jax.experimental import pallas as pl
from jax.experimental.pallas import tpu as pltpu
from jax.experimental.pallas import tpu_sc as plsc

VOCAB = 1000000
D = 32
OUT = 16
B = 16384
SEQ = 50
EPS = 1e-5 * SEQ * SEQ

NC = 2
NS = 16
NW = NC * NS
BPW = B // NW
CHUNK = 128
NCH = BPW // CHUNK


VCHUNK = 65536
TROWS = VCHUNK // 4
FGRID = (VOCAB + VCHUNK - 1) // VCHUNK
VPAD = FGRID * VCHUNK
VSH = VCHUNK.bit_length() - 1
TSH = TROWS.bit_length() - 1


def _fmt_body(tt_ref, o_ref):
    a = tt_ref[...]
    b = jnp.concatenate(
        [a[:, q * TROWS:(q + 1) * TROWS] for q in range(4)], axis=0)
    o_ref[...] = b.T


def _fmt(tableT):
    return pl.pallas_call(
        _fmt_body,
        grid=(FGRID,),
        in_specs=[pl.BlockSpec((D, VCHUNK), lambda i: (0, i))],
        out_specs=pl.BlockSpec((TROWS, 128), lambda i: (i, 0)),
        out_shape=jax.ShapeDtypeStruct((VPAD // 4, 128), jnp.float32),
        compiler_params=pltpu.CompilerParams(
            vmem_limit_bytes=100 * 1024 * 1024),
    )(tableT)


def _sc_pool_body(table_hbm, xp_hbm, out_hbm, idx_v, acc_v, isem, sem):
    wid = lax.axis_index("s") * NC + lax.axis_index("c")

    idx_cp = pltpu.async_copy(xp_hbm.at[:, pl.ds(wid * BPW, BPW)], idx_v,
                              isem)

    def _zero_row(i, _):
        acc_v[i, pl.ds(0, 16)] = jnp.zeros((16,), jnp.float32)
        acc_v[i, pl.ds(16, 16)] = jnp.zeros((16,), jnp.float32)
        return 0
    lax.fori_loop(0, BPW, _zero_row, 0)
    idx_cp.wait()

    for c in range(NCH):
        def _fire(j, _, c=c):
            pltpu.async_copy(
                table_hbm.at[idx_v.at[j, pl.ds(c * CHUNK, CHUNK)]],
                acc_v.at[pl.ds(c * CHUNK, CHUNK)], sem, add=True)
            return 0
        lax.fori_loop(0, SEQ, _fire, 0)

    def _drain(t, _):
        pltpu.make_async_copy(table_hbm.at[idx_v.at[0, pl.ds(0, CHUNK)]],
                              acc_v.at[pl.ds(0, CHUNK)], sem).wait()
        return 0
    lax.fori_loop(0, SEQ * NCH, _drain, 0)

    pltpu.sync_copy(acc_v, out_hbm.at[pl.ds(wid * BPW, BPW)])


def _sc_pool(t4, x):
    mesh = plsc.VectorSubcoreMesh(core_axis_name="c", subcore_axis_name="s")
    return pl.kernel(
        _sc_pool_body,
        out_type=jax.ShapeDtypeStruct((B, D), jnp.float32),
        mesh=mesh,
        scratch_types=[
            pltpu.VMEM((SEQ, BPW), jnp.int32),
            pltpu.VMEM((BPW, D), jnp.float32),
            pltpu.SemaphoreType.DMA,
            pltpu.SemaphoreType.DMA,
        ],
        compiler_params=pltpu.CompilerParams(use_tc_tiling_on_sc=False,
                                             needs_layout_passes=False),
    )(t4, x)


def _head_body(s_ref, g_ref, be_ref, wt_ref, b_ref, o_ref):
    s = s_ref[...]
    mu = jnp.mean(s, axis=1, keepdims=True)
    var = jnp.mean((s - mu) ** 2, axis=1, keepdims=True)
    h = (s - mu) * lax.rsqrt(var + EPS) * g_ref[...] + be_ref[...]
    h = jnp.maximum(h, 0.0)
    o = jnp.dot(h, wt_ref[...],
                preferred_element_type=jnp.float32) + b_ref[...]
    o_ref[...] = o.T


def _head(sums, gamma, beta, Wt, bias):
    blk = 4096
    return pl.pallas_call(
        _head_body,
        grid=(B // blk,),
        in_specs=[
            pl.BlockSpec((blk, D), lambda i: (i, 0)),
            pl.BlockSpec((1, D), lambda i: (0, 0)),
            pl.BlockSpec((1, D), lambda i: (0, 0)),
            pl.BlockSpec((D, OUT), lambda i: (0, 0)),
            pl.BlockSpec((1, OUT), lambda i: (0, 0)),
        ],
        out_specs=pl.BlockSpec((OUT, blk), lambda i: (0, i)),
        out_shape=jax.ShapeDtypeStruct((OUT, B), jnp.float32),
    )(sums, gamma, beta, Wt, bias)


def kernel(x, table, ln_gamma, ln_beta, W, b):
    t4 = _fmt(table.T)
    xt = x.T
    xp = (((xt >> VSH) << VSH) + ((xt & (TROWS - 1)) << 2)
          + ((xt & (VCHUNK - 1)) >> TSH))
    sums = _sc_pool(t4.reshape(VPAD, D), xp)
    return _head(sums, ln_gamma.reshape(1, D), ln_beta.reshape(1, D),
                 W.T, b.reshape(1, OUT)).T
